# Initial kernel scaffold; baseline (speedup 1.0000x reference)
import jax
import jax.numpy as jnp
from jax.experimental import pallas as pl


def kernel(X_sparse, X_dense, tables):
    raise NotImplementedError("write your pallas kernel here")


# speedup vs baseline: 1.1113x; 1.1113x over previous
import functools

import jax
import jax.numpy as jnp
from jax import lax
from jax.experimental import pallas as pl
from jax.experimental.pallas import tpu as pltpu
from jax.experimental.pallas import tpu_sc as plsc

_BATCH = 16384
_NF = 26
_VOCAB = 100000
_EMB = 16
_DENSE = 13
_OUTW = _NF * _EMB + _DENSE

_NC = 2
_NS = 16
_NW = _NC * _NS
_L = 16
_BPW = _BATCH // _NW
_QC = 128
_NQ = _BPW // _QC

_mesh = plsc.VectorSubcoreMesh(core_axis_name="c", subcore_axis_name="s")


@functools.partial(
    pl.kernel,
    mesh=_mesh,
    out_type=jax.ShapeDtypeStruct((_BATCH, _OUTW), jnp.float32),
    scratch_types=[
        pltpu.VMEM((_BPW * _NF,), jnp.int32),
        pltpu.VMEM((_NF * _NQ, _QC), jnp.int32),
        pltpu.VMEM((_NQ, _QC, _EMB), jnp.float32),
        pltpu.VMEM((_BPW, _DENSE), jnp.float32),
        pltpu.SemaphoreType.DMA,
    ],
    compiler_params=pltpu.CompilerParams(
        use_tc_tiling_on_sc=False, needs_layout_passes=False),
)
def _fe(xs_hbm, xd_hbm, tab_hbm, out_hbm, xs_v, idx_v, emb_v, dense_v, sem):
    wid = lax.axis_index("s") * _NC + lax.axis_index("c")
    base = wid * _BPW

    pltpu.sync_copy(xs_hbm.at[pl.ds(base * _NF, _BPW * _NF)], xs_v)
    pltpu.sync_copy(xd_hbm.at[pl.ds(base, _BPW), :], dense_v)

    lanes = lax.iota(jnp.int32, _L)
    for i in range(_NF):
        def body(p, _, i=i):
            pos = (p * _L + lanes) * _NF + i
            vals = plsc.load_gather(xs_v, [pos]) + jnp.int32(i * _VOCAB)
            q = p // (_QC // _L)
            col = (p % (_QC // _L)) * _L
            idx_v[i * _NQ + q, pl.ds(col, _L)] = vals
            return _
        lax.fori_loop(0, _BPW // _L, body, 0)

    for i in range(_NF):
        copies = []
        for q in range(_NQ):
            copies.append(
                pltpu.async_copy(tab_hbm.at[idx_v.at[i * _NQ + q]],
                                 emb_v.at[q], sem))
        for q in range(_NQ):
            copies[q].wait()
        for q in range(_NQ):
            pltpu.sync_copy(
                emb_v.at[q],
                out_hbm.at[pl.ds(base + q * _QC, _QC),
                           pl.ds(i * _EMB, _EMB)])

    pltpu.sync_copy(dense_v,
                    out_hbm.at[pl.ds(base, _BPW), pl.ds(_NF * _EMB, _DENSE)])


def kernel(X_sparse, X_dense, tables):
    xs_flat = X_sparse.astype(jnp.int32).reshape(_BATCH * _NF)
    tab_flat = tables.reshape(_NF * _VOCAB, _EMB)
    return _fe(xs_flat, X_dense, tab_flat)


# speedup vs baseline: 1.1427x; 1.0282x over previous
import functools

import jax
import jax.numpy as jnp
from jax import lax
from jax.experimental import pallas as pl
from jax.experimental.pallas import tpu as pltpu
from jax.experimental.pallas import tpu_sc as plsc

_BATCH = 16384
_NF = 26
_VOCAB = 100000
_EMB = 16
_DENSE = 13
_OUTW = _NF * _EMB + _DENSE

_NC = 2
_NS = 16
_NW = _NC * _NS
_L = 16
_BPW = _BATCH // _NW
_QC = 128
_NQ = _BPW // _QC

_NB = 4
_G = 2

_mesh = plsc.VectorSubcoreMesh(core_axis_name="c", subcore_axis_name="s")


@functools.partial(
    pl.kernel,
    mesh=_mesh,
    out_type=jax.ShapeDtypeStruct((_BATCH, _OUTW), jnp.float32),
    scratch_types=[
        pltpu.VMEM((_BPW * _NF,), jnp.int32),
        pltpu.VMEM((_NF * _NQ, _QC), jnp.int32),
        pltpu.VMEM((_NB, _BPW, _EMB), jnp.float32),
        pltpu.VMEM((_BPW, _DENSE), jnp.float32),
        pltpu.SemaphoreType.DMA((_NB,)),
        pltpu.SemaphoreType.DMA((_NB,)),
    ],
    compiler_params=pltpu.CompilerParams(
        use_tc_tiling_on_sc=False, needs_layout_passes=False),
)
def _fe(xs_hbm, xd_hbm, tab_hbm, out_hbm, xs_v, idx_v, emb_v, dense_v,
        gsem, wsem):
    wid = lax.axis_index("s") * _NC + lax.axis_index("c")
    base = wid * _BPW

    pltpu.sync_copy(xs_hbm.at[pl.ds(base * _NF, _BPW * _NF)], xs_v)
    pltpu.sync_copy(xd_hbm.at[pl.ds(base, _BPW), :], dense_v)

    lanes = lax.iota(jnp.int32, _L)

    def build_idx(i):
        def body(p, _):
            pos = (p * _L + lanes) * _NF + i
            vals = plsc.load_gather(xs_v, [pos]) + jnp.int32(i * _VOCAB)
            q = p // (_QC // _L)
            col = (p % (_QC // _L)) * _L
            idx_v[i * _NQ + q, pl.ds(col, _L)] = vals
            return _
        lax.fori_loop(0, _BPW // _L, body, 0)

    def gather_descs(i, slot):
        return [pltpu.make_async_copy(
                    tab_hbm.at[idx_v.at[i * _NQ + q]],
                    emb_v.at[slot].at[pl.ds(q * _QC, _QC), :], gsem.at[slot])
                for q in range(_NQ)]

    def write_desc(i, slot):
        return pltpu.make_async_copy(
            emb_v.at[slot],
            out_hbm.at[pl.ds(base, _BPW), pl.ds(i * _EMB, _EMB)],
            wsem.at[slot])

    for i in range(_G):
        build_idx(i)
        for d in gather_descs(i, i % _NB):
            d.start()

    for i in range(_NF):
        nxt = i + _G
        if nxt < _NF:
            build_idx(nxt)
            if i >= _NB - _G:
                write_desc(i - (_NB - _G), (i - (_NB - _G)) % _NB).wait()
            for d in gather_descs(nxt, nxt % _NB):
                d.start()
        for d in gather_descs(i, i % _NB):
            d.wait()
        write_desc(i, i % _NB).start()

    pltpu.sync_copy(dense_v,
                    out_hbm.at[pl.ds(base, _BPW), pl.ds(_NF * _EMB, _DENSE)])
    for i in range(_NF - (_NB - _G), _NF):
        write_desc(i, i % _NB).wait()


def kernel(X_sparse, X_dense, tables):
    xs_flat = X_sparse.astype(jnp.int32).reshape(_BATCH * _NF)
    tab_flat = tables.reshape(_NF * _VOCAB, _EMB)
    return _fe(xs_flat, X_dense, tab_flat)


# speedup vs baseline: 3.3577x; 2.9384x over previous
import functools

import jax
import jax.numpy as jnp
from jax import lax
from jax.experimental import pallas as pl
from jax.experimental.pallas import tpu as pltpu
from jax.experimental.pallas import tpu_sc as plsc

_BATCH = 16384
_NF = 26
_VOCAB = 100000
_EMB = 16
_DENSE = 13
_OUTW = _NF * _EMB + _DENSE

_VPAD = 100096
_NG = 52
_NGO = 54
_NBB = _BATCH // 128
_NW = 32
_BPW = _NBB // _NW
_L = 16
_GSTRIDE = _VPAD * 8

_mesh = plsc.VectorSubcoreMesh(core_axis_name="c", subcore_axis_name="s")


@functools.partial(
    pl.kernel,
    mesh=_mesh,
    out_type=jax.ShapeDtypeStruct((_NGO, 128, 1024), jnp.float32),
    scratch_types=[
        pltpu.VMEM((_BPW, 4, 1024), jnp.int32),
        pltpu.VMEM((2, _BPW, 1024), jnp.int32),
        pltpu.VMEM((2, _BPW, 1024), jnp.float32),
        pltpu.VMEM((1024,), jnp.float32),
        pltpu.SemaphoreType.DMA((2,)),
        pltpu.SemaphoreType.DMA((2,)),
    ],
    compiler_params=pltpu.CompilerParams(
        use_tc_tiling_on_sc=False, needs_layout_passes=False),
)
def _fe(tab1d, xs4, xd4, out4, xs_v, idx_v, emb_v, dns_v, gsem, wsem):
    wid = lax.axis_index("s") * 2 + lax.axis_index("c")
    b0 = wid * _BPW
    lanes = lax.iota(jnp.int32, _L)

    for bi in range(_BPW):
        for gf in range(4):
            pltpu.sync_copy(xs4.at[gf, b0 + bi], xs_v.at[bi, gf])

    def build_idx(g, par):
        i = g >> 1
        gf = i >> 3
        f = i & 7
        base = g * _GSTRIDE
        for bi in range(_BPW):
            def chunk(p, carry, bi=bi):
                v = xs_v[bi, gf, pl.ds(f * 128 + p * _L, _L)]
                t = base + ((v >> 7) << 10) + (v & 127)
                for e in range(8):
                    idx_v[par, bi, pl.ds(e * 128 + p * _L, _L)] = (
                        t + e * 128)
                return carry
            lax.fori_loop(0, 8, chunk, 0)

    def gather_descs(par):
        return [pltpu.make_async_copy(
                    tab1d.at[idx_v.at[par, bi]], emb_v.at[par, bi],
                    gsem.at[par])
                for bi in range(_BPW)]

    def write_descs(g, par):
        return [pltpu.make_async_copy(
                    emb_v.at[par, bi], out4.at[g, b0 + bi], wsem.at[par])
                for bi in range(_BPW)]

    def loop_body(g, carry):
        par = g & 1

        @pl.when(g >= 2)
        def _():
            for d in write_descs(g - 2, par):
                d.wait()

        build_idx(g, par)
        for d in gather_descs(par):
            d.start()

        @pl.when(g >= 1)
        def _():
            for d in gather_descs(1 - par):
                d.wait()
            for d in write_descs(g - 1, 1 - par):
                d.start()
        return carry

    lax.fori_loop(0, _NG, loop_body, 0)

    last_par = (_NG - 1) & 1
    for d in write_descs(_NG - 2, (_NG - 2) & 1):
        d.wait()
    for d in gather_descs(last_par):
        d.wait()
    for d in write_descs(_NG - 1, last_par):
        d.start()

    for bi in range(_BPW):
        for k in range(2):
            pltpu.sync_copy(xd4.at[k, b0 + bi], dns_v)
            pltpu.sync_copy(dns_v, out4.at[_NG + k, b0 + bi])

    for d in write_descs(_NG - 1, last_par):
        d.wait()


def kernel(X_sparse, X_dense, tables):
    tabP = jnp.pad(tables, ((0, 0), (0, _VPAD - _VOCAB), (0, 0)))
    tab1d = (tabP.reshape(_NF, _VPAD // 128, 128, 2, 8)
             .transpose(0, 3, 1, 4, 2).reshape(_NF * _EMB * _VPAD))

    xs4 = (jnp.pad(X_sparse, ((0, 0), (0, 6))).astype(jnp.int32)
           .reshape(128, 128, 4, 8).transpose(2, 0, 3, 1)
           .reshape(4, 128, 1024))
    xd4 = (jnp.pad(X_dense, ((0, 0), (0, 3)))
           .reshape(128, 128, 2, 8).transpose(2, 0, 3, 1)
           .reshape(2, 128, 1024))

    o4 = _fe(tab1d, xs4, xd4)
    return (o4.reshape(_NGO, 128, 8, 128).transpose(1, 3, 0, 2)
            .reshape(_BATCH, _NGO * 8)[:, :_OUTW])


# speedup vs baseline: 3.4911x; 1.0397x over previous
import functools

import jax
import jax.numpy as jnp
from jax import lax
from jax.experimental import pallas as pl
from jax.experimental.pallas import tpu as pltpu
from jax.experimental.pallas import tpu_sc as plsc

_BATCH = 16384
_NF = 26
_VOCAB = 100000
_EMB = 16
_DENSE = 13
_OUTW = _NF * _EMB + _DENSE

_VPAD = 100096
_NG = 52
_NGO = 54
_NBB = _BATCH // 128
_NW = 32
_BPW = _NBB // _NW
_L = 16
_GSTRIDE = _VPAD * 8

_mesh = plsc.VectorSubcoreMesh(core_axis_name="c", subcore_axis_name="s")


@functools.partial(
    pl.kernel,
    mesh=_mesh,
    out_type=jax.ShapeDtypeStruct((_NGO, 128, 1024), jnp.float32),
    scratch_types=[
        pltpu.VMEM((_BPW, 4, 1024), jnp.int32),
        pltpu.VMEM((4, _BPW * 1024), jnp.int32),
        pltpu.VMEM((4, _BPW * 1024), jnp.float32),
        pltpu.VMEM((1024,), jnp.float32),
        pltpu.SemaphoreType.DMA((4,)),
        pltpu.SemaphoreType.DMA((4,)),
    ],
    compiler_params=pltpu.CompilerParams(
        use_tc_tiling_on_sc=False, needs_layout_passes=False),
)
def _fe(tab1d, xs4, xd4, out4, xs_v, idx_v, emb_v, dns_v, gsem, wsem):
    wid = lax.axis_index("s") * 2 + lax.axis_index("c")
    b0 = wid * _BPW
    lanes = lax.iota(jnp.int32, _L)

    for bi in range(_BPW):
        for gf in range(4):
            pltpu.sync_copy(xs4.at[gf, b0 + bi], xs_v.at[bi, gf])

    def build_idx(g, par):
        i = g >> 1
        gf = i >> 3
        f = i & 7
        base = g * _GSTRIDE
        for bi in range(_BPW):
            def chunk(p, carry, bi=bi):
                v = xs_v[bi, gf, pl.ds(f * 128 + p * _L, _L)]
                t = base + ((v >> 7) << 10) + (v & 127)
                for e in range(8):
                    idx_v[par, pl.ds(bi * 1024 + e * 128 + p * _L, _L)] = (
                        t + e * 128)
                return carry
            lax.fori_loop(0, 8, chunk, 0)

    def gather_desc(par):
        return pltpu.make_async_copy(
            tab1d.at[idx_v.at[par]], emb_v.at[par], gsem.at[par])

    def write_descs(g, par):
        return [pltpu.make_async_copy(
                    emb_v.at[par, pl.ds(bi * 1024, 1024)],
                    out4.at[g, b0 + bi], wsem.at[par])
                for bi in range(_BPW)]

    def loop_body(g, carry):
        par = g & 3

        @pl.when(g >= 4)
        def _():
            for d in write_descs(g - 4, par):
                d.wait()

        build_idx(g, par)
        gather_desc(par).start()

        @pl.when(g >= 1)
        def _():
            par1 = (g - 1) & 3
            gather_desc(par1).wait()
            for d in write_descs(g - 1, par1):
                d.start()
        return carry

    lax.fori_loop(0, _NG, loop_body, 0)

    last_par = (_NG - 1) & 3
    gather_desc(last_par).wait()
    for d in write_descs(_NG - 1, last_par):
        d.start()
    for t in range(_NG - 4, _NG - 1):
        for d in write_descs(t, t & 3):
            d.wait()

    for bi in range(_BPW):
        for k in range(2):
            pltpu.sync_copy(xd4.at[k, b0 + bi], dns_v)
            pltpu.sync_copy(dns_v, out4.at[_NG + k, b0 + bi])

    for d in write_descs(_NG - 1, last_par):
        d.wait()


def kernel(X_sparse, X_dense, tables):
    tabP = jnp.pad(tables, ((0, 0), (0, _VPAD - _VOCAB), (0, 0)))
    tab1d = (tabP.reshape(_NF, _VPAD // 128, 128, 2, 8)
             .transpose(0, 3, 1, 4, 2).reshape(_NF * _EMB * _VPAD))

    xs4 = (jnp.pad(X_sparse, ((0, 0), (0, 6))).astype(jnp.int32)
           .reshape(128, 128, 4, 8).transpose(2, 0, 3, 1)
           .reshape(4, 128, 1024))
    xd4 = (jnp.pad(X_dense, ((0, 0), (0, 3)))
           .reshape(128, 128, 2, 8).transpose(2, 0, 3, 1)
           .reshape(2, 128, 1024))

    o4 = _fe(tab1d, xs4, xd4)
    return (o4.reshape(_NGO, 128, 8, 128).transpose(1, 3, 0, 2)
            .reshape(_BATCH, _NGO * 8)[:, :_OUTW])
